# baseline (speedup 1.0000x reference)
import functools

import jax
import jax.numpy as jnp
from jax import lax
from jax.experimental import pallas as pl
from jax.experimental.pallas import tpu as pltpu

_B = 8
_N = 16384
_S = 1024
_K = 128
_NCH = 8

_BIG = 1 << 30


def _fps_body(init_ref, x0_ref, x1_ref, x2_ref, s0_ref, s1_ref, s2_ref, dist_ref):
    lane = lax.broadcasted_iota(jnp.int32, (_B, _K), 1)
    iota_s = lax.broadcasted_iota(jnp.int32, (_B, _S), 1)

    dist_ref[...] = jnp.full((_K * _B, _K), 1e10, jnp.float32)
    riota3 = (
        lax.broadcasted_iota(jnp.int32, (_K, _B, _K), 0) * _K
        + lax.broadcasted_iota(jnp.int32, (_K, _B, _K), 2)
    )
    init3 = init_ref[...].reshape(1, _B, 1)
    mask3 = riota3 == init3

    def _gather0(ref):
        v = jnp.where(mask3, ref[...].reshape(_K, _B, _K), 0.0)
        return jnp.sum(jnp.sum(v, axis=0), axis=1, keepdims=True)

    c0_0 = _gather0(x0_ref)
    c1_0 = _gather0(x1_ref)
    c2_0 = _gather0(x2_ref)

    def body(i, state):
        c0, c1, c2, a0, a1, a2 = state
        rec = iota_s == i
        a0 = jnp.where(rec, c0, a0)
        a1 = jnp.where(rec, c1, a1)
        a2 = jnp.where(rec, c2, a2)

        ninf = jnp.full((_B, _K), -jnp.inf, jnp.float32)
        zf = jnp.zeros((_B, _K), jnp.float32)
        zi = jnp.zeros((_B, _K), jnp.int32)
        bv = [ninf] * _NCH
        bk = [zi] * _NCH
        p0 = [zf] * _NCH
        p1 = [zf] * _NCH
        p2 = [zf] * _NCH

        for k in range(_K):
            sl = slice(k * _B, (k + 1) * _B)
            xv0 = x0_ref[sl, :]
            xv1 = x1_ref[sl, :]
            xv2 = x2_ref[sl, :]
            d = ((xv0 - c0) ** 2 + (xv2 - c2) ** 2) + (xv1 - c1) ** 2
            nd = jnp.minimum(dist_ref[sl, :], d)
            dist_ref[sl, :] = nd
            j = k % _NCH
            upd = nd > bv[j]
            bv[j] = jnp.where(upd, nd, bv[j])
            bk[j] = jnp.where(upd, k, bk[j])
            p0[j] = jnp.where(upd, xv0, p0[j])
            p1[j] = jnp.where(upd, xv1, p1[j])
            p2[j] = jnp.where(upd, xv2, p2[j])

        mv, mk, m0, m1, m2 = bv[0], bk[0], p0[0], p1[0], p2[0]
        for j in range(1, _NCH):
            tb = (bv[j] > mv) | ((bv[j] == mv) & (bk[j] < mk))
            mv = jnp.where(tb, bv[j], mv)
            mk = jnp.where(tb, bk[j], mk)
            m0 = jnp.where(tb, p0[j], m0)
            m1 = jnp.where(tb, p1[j], m1)
            m2 = jnp.where(tb, p2[j], m2)

        m = jnp.max(mv, axis=1, keepdims=True)
        enc = jnp.where(mv == m, mk * _K + lane, _BIG)
        nstar = jnp.min(enc, axis=1, keepdims=True)
        lsel = enc == nstar
        nc0 = jnp.sum(jnp.where(lsel, m0, 0.0), axis=1, keepdims=True)
        nc1 = jnp.sum(jnp.where(lsel, m1, 0.0), axis=1, keepdims=True)
        nc2 = jnp.sum(jnp.where(lsel, m2, 0.0), axis=1, keepdims=True)
        return nc0, nc1, nc2, a0, a1, a2

    z = jnp.zeros((_B, _S), jnp.float32)
    _, _, _, a0, a1, a2 = lax.fori_loop(0, _S, body, (c0_0, c1_0, c2_0, z, z, z))
    s0_ref[...] = a0
    s1_ref[...] = a1
    s2_ref[...] = a2


def _mlp_body(sp_ref, w1_ref, b1_ref, w2_ref, b2_ref, w3_ref, b3_ref, out_ref):
    sp = sp_ref[...]
    w1 = w1_ref[...]
    h = (
        sp[:, 0:1] * w1[0:1, :]
        + sp[:, 1:2] * w1[1:2, :]
        + sp[:, 2:3] * w1[2:3, :]
        + b1_ref[...]
    )
    h = jnp.maximum(h, 0.0)
    h = jnp.dot(h, w2_ref[...], preferred_element_type=jnp.float32) + b2_ref[...]
    h = jnp.maximum(h, 0.0)
    out_ref[...] = (
        jnp.dot(h, w3_ref[...], preferred_element_type=jnp.float32) + b3_ref[...]
    )


@functools.partial(jax.jit, static_argnums=())
def kernel(x, W1, b1, W2, b2, W3, b3, init_centroid):
    xr = x.reshape(_B, _K, _K, 3).transpose(1, 0, 2, 3)
    x0 = xr[..., 0].reshape(_K * _B, _K)
    x1 = xr[..., 1].reshape(_K * _B, _K)
    x2 = xr[..., 2].reshape(_K * _B, _K)
    init = init_centroid.astype(jnp.int32).reshape(_B, 1)

    s0, s1, s2 = pl.pallas_call(
        _fps_body,
        out_shape=[jax.ShapeDtypeStruct((_B, _S), jnp.float32)] * 3,
        scratch_shapes=[pltpu.VMEM((_K * _B, _K), jnp.float32)],
    )(init, x0, x1, x2)

    sampled = jnp.stack([s0, s1, s2], axis=-1)

    feats = pl.pallas_call(
        _mlp_body,
        out_shape=jax.ShapeDtypeStruct((_B * _S, 256), jnp.float32),
    )(
        sampled.reshape(_B * _S, 3),
        W1,
        b1.reshape(1, 64),
        W2,
        b2.reshape(1, 128),
        W3,
        b3.reshape(1, 256),
    )
    return sampled, feats.reshape(_B, _S, 256)


# speedup vs baseline: 1.0715x; 1.0715x over previous
import functools

import jax
import jax.numpy as jnp
from jax import lax
from jax.experimental import pallas as pl
from jax.experimental.pallas import tpu as pltpu

_B = 8
_N = 16384
_S = 1024
_K = 128
_NCH = 2

_BIG = 1 << 30


def _fps_body(init_ref, x0_ref, x1_ref, x2_ref, s0_ref, s1_ref, s2_ref, dist_ref):
    lane = lax.broadcasted_iota(jnp.int32, (_B, _K), 1)
    iota_s = lax.broadcasted_iota(jnp.int32, (_B, _S), 1)

    dist_ref[...] = jnp.full((_K * _B, _K), 1e10, jnp.float32)
    riota3 = (
        lax.broadcasted_iota(jnp.int32, (_K, _B, _K), 0) * _K
        + lax.broadcasted_iota(jnp.int32, (_K, _B, _K), 2)
    )
    init3 = init_ref[...].reshape(1, _B, 1)
    mask3 = riota3 == init3

    def _gather0(ref):
        v = jnp.where(mask3, ref[...].reshape(_K, _B, _K), 0.0)
        return jnp.sum(jnp.sum(v, axis=0), axis=1, keepdims=True)

    c0_0 = _gather0(x0_ref)
    c1_0 = _gather0(x1_ref)
    c2_0 = _gather0(x2_ref)

    def body(i, state):
        c0, c1, c2, a0, a1, a2 = state
        rec = iota_s == i
        a0 = jnp.where(rec, c0, a0)
        a1 = jnp.where(rec, c1, a1)
        a2 = jnp.where(rec, c2, a2)

        ninf = jnp.full((_B, _K), -jnp.inf, jnp.float32)
        zf = jnp.zeros((_B, _K), jnp.float32)
        zi = jnp.zeros((_B, _K), jnp.int32)
        bv = [ninf] * _NCH
        bk = [zi] * _NCH
        p0 = [zf] * _NCH
        p1 = [zf] * _NCH
        p2 = [zf] * _NCH

        for k in range(_K):
            sl = slice(k * _B, (k + 1) * _B)
            xv0 = x0_ref[sl, :]
            xv1 = x1_ref[sl, :]
            xv2 = x2_ref[sl, :]
            d = ((xv0 - c0) ** 2 + (xv2 - c2) ** 2) + (xv1 - c1) ** 2
            nd = jnp.minimum(dist_ref[sl, :], d)
            dist_ref[sl, :] = nd
            j = k % _NCH
            upd = nd > bv[j]
            bv[j] = jnp.where(upd, nd, bv[j])
            bk[j] = jnp.where(upd, k, bk[j])
            p0[j] = jnp.where(upd, xv0, p0[j])
            p1[j] = jnp.where(upd, xv1, p1[j])
            p2[j] = jnp.where(upd, xv2, p2[j])

        mv, mk, m0, m1, m2 = bv[0], bk[0], p0[0], p1[0], p2[0]
        for j in range(1, _NCH):
            tb = (bv[j] > mv) | ((bv[j] == mv) & (bk[j] < mk))
            mv = jnp.where(tb, bv[j], mv)
            mk = jnp.where(tb, bk[j], mk)
            m0 = jnp.where(tb, p0[j], m0)
            m1 = jnp.where(tb, p1[j], m1)
            m2 = jnp.where(tb, p2[j], m2)

        m = jnp.max(mv, axis=1, keepdims=True)
        enc = jnp.where(mv == m, mk * _K + lane, _BIG)
        nstar = jnp.min(enc, axis=1, keepdims=True)
        lsel = enc == nstar
        nc0 = jnp.sum(jnp.where(lsel, m0, 0.0), axis=1, keepdims=True)
        nc1 = jnp.sum(jnp.where(lsel, m1, 0.0), axis=1, keepdims=True)
        nc2 = jnp.sum(jnp.where(lsel, m2, 0.0), axis=1, keepdims=True)
        return nc0, nc1, nc2, a0, a1, a2

    z = jnp.zeros((_B, _S), jnp.float32)
    _, _, _, a0, a1, a2 = lax.fori_loop(0, _S, body, (c0_0, c1_0, c2_0, z, z, z))
    s0_ref[...] = a0
    s1_ref[...] = a1
    s2_ref[...] = a2


def _mlp_body(sp_ref, w1_ref, b1_ref, w2_ref, b2_ref, w3_ref, b3_ref, out_ref):
    sp = sp_ref[...]
    w1 = w1_ref[...]
    h = (
        sp[:, 0:1] * w1[0:1, :]
        + sp[:, 1:2] * w1[1:2, :]
        + sp[:, 2:3] * w1[2:3, :]
        + b1_ref[...]
    )
    h = jnp.maximum(h, 0.0)
    h = jnp.dot(h, w2_ref[...], preferred_element_type=jnp.float32) + b2_ref[...]
    h = jnp.maximum(h, 0.0)
    out_ref[...] = (
        jnp.dot(h, w3_ref[...], preferred_element_type=jnp.float32) + b3_ref[...]
    )


@functools.partial(jax.jit, static_argnums=())
def kernel(x, W1, b1, W2, b2, W3, b3, init_centroid):
    xr = x.reshape(_B, _K, _K, 3).transpose(1, 0, 2, 3)
    x0 = xr[..., 0].reshape(_K * _B, _K)
    x1 = xr[..., 1].reshape(_K * _B, _K)
    x2 = xr[..., 2].reshape(_K * _B, _K)
    init = init_centroid.astype(jnp.int32).reshape(_B, 1)

    s0, s1, s2 = pl.pallas_call(
        _fps_body,
        out_shape=[jax.ShapeDtypeStruct((_B, _S), jnp.float32)] * 3,
        scratch_shapes=[pltpu.VMEM((_K * _B, _K), jnp.float32)],
    )(init, x0, x1, x2)

    sampled = jnp.stack([s0, s1, s2], axis=-1)

    feats = pl.pallas_call(
        _mlp_body,
        out_shape=jax.ShapeDtypeStruct((_B * _S, 256), jnp.float32),
    )(
        sampled.reshape(_B * _S, 3),
        W1,
        b1.reshape(1, 64),
        W2,
        b2.reshape(1, 128),
        W3,
        b3.reshape(1, 256),
    )
    return sampled, feats.reshape(_B, _S, 256)
